# baseline (speedup 1.0000x reference)
import functools

import jax
import jax.numpy as jnp
from jax import lax
from jax.experimental import pallas as pl
from jax.experimental.pallas import tpu as pltpu
from jax.experimental.pallas import tpu_sc as plsc

L = 16
CHUNK = 8192
NB = 8192
WMAX = 15


def _interp_kernel(B, K, N, NC, NS):
    NW = NC * NS
    wpr = NW // B
    qw = N // wpr
    n_chunks = qw // CHUNK
    eps = float(jnp.finfo(jnp.float32).eps)
    nb_pad = NB + 4 * L

    mesh = plsc.VectorSubcoreMesh(
        core_axis_name="c", subcore_axis_name="s", num_cores=NC, num_subcores=NS
    )

    @functools.partial(
        pl.kernel,
        out_type=jax.ShapeDtypeStruct((B, N), jnp.float32),
        mesh=mesh,
        compiler_params=pltpu.CompilerParams(needs_layout_passes=False),
        scratch_types=[
            pltpu.VMEM((K + L,), jnp.float32),
            pltpu.VMEM((K,), jnp.float32),
            pltpu.VMEM((K,), jnp.float32),
            pltpu.VMEM((nb_pad,), jnp.int32),
            pltpu.VMEM((CHUNK,), jnp.float32),
            pltpu.VMEM((CHUNK,), jnp.float32),
            pltpu.VMEM((CHUNK,), jnp.float32),
            pltpu.VMEM((CHUNK,), jnp.float32),
            pltpu.SemaphoreType.DMA,
            pltpu.SemaphoreType.DMA,
            pltpu.SemaphoreType.DMA,
            pltpu.SemaphoreType.DMA,
        ],
    )
    def body(xq_hbm, x_hbm, y_hbm, out_hbm, xr, yr, sr, lut, qb0, qb1,
             ob0, ob1, sin0, sin1, sout0, sout1):
        wid = lax.axis_index("s") * NC + lax.axis_index("c")
        row = wid // wpr
        qoff = (wid % wpr) * qw

        pltpu.sync_copy(x_hbm.at[row], xr.at[pl.ds(0, K)])
        pltpu.sync_copy(y_hbm.at[row], yr)
        xr[pl.ds(K, L)] = jnp.full((L,), jnp.inf, jnp.float32)

        iota = lax.iota(jnp.int32, L)
        zeros = jnp.zeros((L,), jnp.int32)

        @plsc.parallel_loop(0, K, step=L, unroll=4)
        def _(base):
            idx1 = jnp.minimum(base + iota + 1, K - 1)
            x0 = xr[pl.ds(base, L)]
            y0 = yr[pl.ds(base, L)]
            x1 = plsc.load_gather(xr, [idx1])
            y1 = plsc.load_gather(yr, [idx1])
            sr[pl.ds(base, L)] = (y1 - y0) / (eps + (x1 - x0))

        @plsc.parallel_loop(0, K, step=L, unroll=4)
        def _(base):
            yr[pl.ds(base, L)] = (
                yr[pl.ds(base, L)] - sr[pl.ds(base, L)] * xr[pl.ds(base, L)]
            )

        def count_below(q):
            pos = zeros
            s = K // 2
            while s >= 1:
                xv = plsc.load_gather(xr, [pos + (s - 1)])
                pos = jnp.where(xv < q, pos + s, pos)
                s //= 2
            return pos

        head = xr[pl.ds(0, L)]
        tail = xr[pl.ds(K - L, L)]
        xmin_v = jnp.full((L,), head[0], jnp.float32)
        rng = jnp.full((L,), tail[L - 1] - head[0], jnp.float32)
        inv_v = rng * (1.0 / NB)
        sc_v = NB / jnp.maximum(rng, 1e-30)

        @plsc.parallel_loop(0, nb_pad, step=L, unroll=4)
        def _(bb):
            b = bb + iota
            edge = xmin_v + (b - 1).astype(jnp.float32) * inv_v
            cnt = count_below(edge)
            lut[pl.ds(bb, L)] = jnp.where(b >= NB, K, cnt)

        def wb(i, carry):
            b = i * L + iota
            hi = plsc.load_gather(lut, [b + 3])
            lo = plsc.load_gather(lut, [b])
            return jnp.maximum(carry, hi - lo)
        wmax = jnp.max(lax.fori_loop(0, NB // L, wb, zeros))

        def make_search(steps):
            def search_fast(q):
                t = jnp.clip((q - xmin_v) * sc_v, 0.0, float(NB - 1))
                b = t.astype(jnp.int32)
                pos = plsc.load_gather(lut, [b])
                for s in steps:
                    cand = pos + s
                    xv = plsc.load_gather(xr, [cand - 1])
                    pos = jnp.where(xv < q, cand, pos)
                return pos
            return search_fast

        sin = (sin0, sin1)
        sout = (sout0, sout1)
        qb = (qb0, qb1)
        ob = (ob0, ob1)

        def in_copy(c, p):
            return pltpu.make_async_copy(
                xq_hbm.at[row, pl.ds(qoff + c * CHUNK, CHUNK)], qb[p], sin[p]
            )

        def out_copy(c, p):
            return pltpu.make_async_copy(
                ob[p], out_hbm.at[row, pl.ds(qoff + c * CHUNK, CHUNK)], sout[p]
            )

        def chunk_loop(search_fn):
            def compute(qbp, obp):
                @plsc.parallel_loop(0, CHUNK, step=L, unroll=8)
                def _(i):
                    q = qbp[pl.ds(i, L)]
                    ind = jnp.clip(search_fn(q) - 1, 0, K - 2)
                    av = plsc.load_gather(yr, [ind])
                    sv = plsc.load_gather(sr, [ind])
                    obp[pl.ds(i, L)] = av + sv * q

            in_copy(0, 0).start()
            in_copy(1, 1).start()

            def pair(i, carry):
                c0 = 2 * i
                for p in range(2):
                    in_copy(c0 + p, p).wait()

                    @pl.when(i > 0)
                    def _():
                        out_copy(c0 + p - 2, p).wait()

                    compute(qb[p], ob[p])
                    out_copy(c0 + p, p).start()

                    @pl.when(c0 + p + 2 < n_chunks)
                    def _():
                        in_copy(c0 + p + 2, p).start()

                return carry

            lax.fori_loop(0, n_chunks // 2, pair, 0)
            out_copy(n_chunks - 2, 0).wait()
            out_copy(n_chunks - 1, 1).wait()

        @pl.when(wmax <= 7)
        def _():
            chunk_loop(make_search((4, 2, 1)))

        @pl.when((wmax > 7) & (wmax <= WMAX))
        def _():
            chunk_loop(make_search((8, 4, 2, 1)))

        @pl.when(wmax > WMAX)
        def _():
            chunk_loop(count_below)

    return body


@jax.jit
def kernel(x_new, x, y, T):
    del T
    B, N = x_new.shape
    K = x.shape[1]
    info = plsc.get_sparse_core_info()
    fn = _interp_kernel(B, K, N, info.num_cores, info.num_subcores)
    return fn(x_new, x, y)


# speedup vs baseline: 1.1443x; 1.1443x over previous
import functools

import jax
import jax.numpy as jnp
from jax import lax
from jax.experimental import pallas as pl
from jax.experimental.pallas import tpu as pltpu
from jax.experimental.pallas import tpu_sc as plsc

L = 16
CHUNK = 8192
NB = 4096
WMAX = 15


def _interp_kernel(B, K, N, NC, NS):
    NW = NC * NS
    wpr = NW // B
    qw = N // wpr
    n_chunks = qw // CHUNK
    eps = float(jnp.finfo(jnp.float32).eps)
    nb_pad = NB + 4 * L

    mesh = plsc.VectorSubcoreMesh(
        core_axis_name="c", subcore_axis_name="s", num_cores=NC, num_subcores=NS
    )

    @functools.partial(
        pl.kernel,
        out_type=jax.ShapeDtypeStruct((B, N), jnp.float32),
        mesh=mesh,
        compiler_params=pltpu.CompilerParams(needs_layout_passes=False),
        scratch_types=[
            pltpu.VMEM((K + L,), jnp.float32),
            pltpu.VMEM((K,), jnp.float32),
            pltpu.VMEM((K,), jnp.float32),
            pltpu.VMEM((nb_pad,), jnp.int32),
            pltpu.VMEM((CHUNK,), jnp.float32),
            pltpu.VMEM((CHUNK,), jnp.float32),
            pltpu.VMEM((CHUNK,), jnp.float32),
            pltpu.VMEM((CHUNK,), jnp.float32),
            pltpu.SemaphoreType.DMA,
            pltpu.SemaphoreType.DMA,
            pltpu.SemaphoreType.DMA,
            pltpu.SemaphoreType.DMA,
        ],
    )
    def body(xq_hbm, x_hbm, y_hbm, out_hbm, xr, yr, sr, lut, qb0, qb1,
             ob0, ob1, sin0, sin1, sout0, sout1):
        wid = lax.axis_index("s") * NC + lax.axis_index("c")
        row = wid // wpr
        qoff = (wid % wpr) * qw

        pltpu.sync_copy(x_hbm.at[row], xr.at[pl.ds(0, K)])
        pltpu.sync_copy(y_hbm.at[row], yr)
        xr[pl.ds(K, L)] = jnp.full((L,), jnp.inf, jnp.float32)

        iota = lax.iota(jnp.int32, L)
        zeros = jnp.zeros((L,), jnp.int32)

        @plsc.parallel_loop(0, K, step=L, unroll=4)
        def _(base):
            idx1 = jnp.minimum(base + iota + 1, K - 1)
            x0 = xr[pl.ds(base, L)]
            y0 = yr[pl.ds(base, L)]
            x1 = plsc.load_gather(xr, [idx1])
            y1 = plsc.load_gather(yr, [idx1])
            sr[pl.ds(base, L)] = (y1 - y0) / (eps + (x1 - x0))

        @plsc.parallel_loop(0, K, step=L, unroll=4)
        def _(base):
            yr[pl.ds(base, L)] = (
                yr[pl.ds(base, L)] - sr[pl.ds(base, L)] * xr[pl.ds(base, L)]
            )

        def count_below(q):
            pos = zeros
            s = K // 2
            while s >= 1:
                xv = plsc.load_gather(xr, [pos + (s - 1)])
                pos = jnp.where(xv < q, pos + s, pos)
                s //= 2
            return pos

        head = xr[pl.ds(0, L)]
        tail = xr[pl.ds(K - L, L)]
        xmin_v = jnp.full((L,), head[0], jnp.float32)
        rng = jnp.full((L,), tail[L - 1] - head[0], jnp.float32)
        inv_v = rng * (1.0 / NB)
        sc_v = NB / jnp.maximum(rng, 1e-30)

        @plsc.parallel_loop(0, nb_pad, step=L, unroll=4)
        def _(bb):
            b = bb + iota
            edge = xmin_v + (b - 1).astype(jnp.float32) * inv_v
            cnt = count_below(edge)
            lut[pl.ds(bb, L)] = jnp.where(b >= NB, K, cnt)

        def wb(i, carry):
            b = i * L + iota
            hi = plsc.load_gather(lut, [b + 3])
            lo = plsc.load_gather(lut, [b])
            return jnp.maximum(carry, hi - lo)
        wmax = jnp.max(lax.fori_loop(0, NB // L, wb, zeros))

        def make_search(steps):
            def search_fast(q):
                t = jnp.clip((q - xmin_v) * sc_v, 0.0, float(NB - 1))
                b = t.astype(jnp.int32)
                pos = plsc.load_gather(lut, [b])
                for s in steps:
                    cand = pos + s
                    xv = plsc.load_gather(xr, [cand - 1])
                    pos = jnp.where(xv < q, cand, pos)
                return pos
            return search_fast

        sin = (sin0, sin1)
        sout = (sout0, sout1)
        qb = (qb0, qb1)
        ob = (ob0, ob1)

        def in_copy(c, p):
            return pltpu.make_async_copy(
                xq_hbm.at[row, pl.ds(qoff + c * CHUNK, CHUNK)], qb[p], sin[p]
            )

        def out_copy(c, p):
            return pltpu.make_async_copy(
                ob[p], out_hbm.at[row, pl.ds(qoff + c * CHUNK, CHUNK)], sout[p]
            )

        def chunk_loop(search_fn):
            def compute(qbp, obp):
                @plsc.parallel_loop(0, CHUNK, step=L, unroll=4)
                def _(i):
                    q = qbp[pl.ds(i, L)]
                    ind = jnp.clip(search_fn(q) - 1, 0, K - 2)
                    av = plsc.load_gather(yr, [ind])
                    sv = plsc.load_gather(sr, [ind])
                    obp[pl.ds(i, L)] = av + sv * q

            in_copy(0, 0).start()
            in_copy(1, 1).start()

            def pair(i, carry):
                c0 = 2 * i
                for p in range(2):
                    in_copy(c0 + p, p).wait()

                    @pl.when(i > 0)
                    def _():
                        out_copy(c0 + p - 2, p).wait()

                    compute(qb[p], ob[p])
                    out_copy(c0 + p, p).start()

                    @pl.when(c0 + p + 2 < n_chunks)
                    def _():
                        in_copy(c0 + p + 2, p).start()

                return carry

            lax.fori_loop(0, n_chunks // 2, pair, 0)
            out_copy(n_chunks - 2, 0).wait()
            out_copy(n_chunks - 1, 1).wait()

        @pl.when(wmax <= 7)
        def _():
            chunk_loop(make_search((4, 2, 1)))

        @pl.when((wmax > 7) & (wmax <= WMAX))
        def _():
            chunk_loop(make_search((8, 4, 2, 1)))

        @pl.when(wmax > WMAX)
        def _():
            chunk_loop(count_below)

    return body


@jax.jit
def kernel(x_new, x, y, T):
    del T
    B, N = x_new.shape
    K = x.shape[1]
    info = plsc.get_sparse_core_info()
    fn = _interp_kernel(B, K, N, info.num_cores, info.num_subcores)
    return fn(x_new, x, y)
